# baseline (speedup 1.0000x reference)
import functools

import jax
import jax.numpy as jnp
from jax import lax
from jax.experimental import pallas as pl
from jax.experimental.pallas import tpu as pltpu
from jax.experimental.pallas import tpu_sc as plsc

_NC = 2
_NS = 16
_NTILES = _NC * _NS
_CHUNK = 128
_CNTW = 16


def _make_sc_segsum(n_rows, n_acc, tile_e, nch, width):
    rows_per_tile = n_acc // _NS
    stage_per_tile = n_rows // _NS
    full_ch = tile_e // _CHUNK
    rem = tile_e - full_ch * _CHUNK
    mesh = plsc.VectorSubcoreMesh(core_axis_name="c", subcore_axis_name="s")

    out_type = jax.ShapeDtypeStruct((_NC, n_acc, width), jnp.float32)
    scratch = [
        pltpu.VMEM((nch, 2, _CHUNK), jnp.int32),
        pltpu.VMEM((_CHUNK, width), jnp.float32),
        pltpu.VMEM((_CHUNK, width), jnp.float32),
        pltpu.VMEM((_CHUNK, width), jnp.float32),
        pltpu.VMEM((_CHUNK, width), jnp.float32),
        pltpu.VMEM_SHARED((n_rows, width), jnp.float32),
        pltpu.VMEM_SHARED((n_acc, width), jnp.float32),
        pltpu.SemaphoreType.DMA,
        pltpu.SemaphoreType.DMA,
        pltpu.SemaphoreType.DMA,
        pltpu.SemaphoreType.DMA,
        pltpu.SemaphoreType.DMA,
        pltpu.SemaphoreType.DMA,
        pltpu.SemaphoreType.DMA,
        pltpu.SemaphoreType.DMA,
        pltpu.SemaphoreType.DMA,
    ]

    def body(y_hbm, ei_hbm, acc_out,
             idx_v, rows0, rows1, rows2, rows3, y_sh, acc_sh,
             sg0, sg1, sg2, sg3, sc0, sc1, sc2, sc3, ss):
        rows = (rows0, rows1, rows2, rows3)
        sg = (sg0, sg1, sg2, sg3)
        sc = (sc0, sc1, sc2, sc3)
        c = lax.axis_index("c")
        s = lax.axis_index("s")
        tile = c * _NS + s
        e0 = tile * tile_e

        def idx_fire(i, _):
            pltpu.async_copy(
                ei_hbm.at[:, pl.ds(e0 + i * _CHUNK, _CHUNK)], idx_v.at[i], ss)
            return 0

        lax.fori_loop(0, full_ch, idx_fire, 0)

        yrow0 = s * stage_per_tile
        pltpu.async_copy(y_hbm.at[pl.ds(yrow0, stage_per_tile)],
                         y_sh.at[pl.ds(yrow0, stage_per_tile)], sg1)

        z16 = jnp.zeros((16,), jnp.float32)
        wq = width // 16

        def zbody(i, _):
            rows0[i // wq, pl.ds((i % wq) * 16, 16)] = z16
            return 0

        lax.fori_loop(0, _CHUNK * wq, zbody, 0)
        row0 = s * rows_per_tile
        zcopies = []
        off = 0
        k = 0
        while off < rows_per_tile:
            sz = min(_CHUNK, rows_per_tile - off)
            zcopies.append(pltpu.async_copy(
                rows0.at[pl.ds(0, sz)], acc_sh.at[pl.ds(row0 + off, sz)],
                sc[k % 4]))
            off += sz
            k += 1

        dummy = jnp.full((16,), n_rows, jnp.int32)
        zi16 = jnp.zeros((16,), jnp.int32)
        for i in range(full_ch, nch):
            for k in range(_CHUNK // 16):
                idx_v[i, 0, pl.ds(k * 16, 16)] = zi16
                idx_v[i, 1, pl.ds(k * 16, 16)] = dummy
        if rem:
            pltpu.async_copy(
                ei_hbm.at[:, pl.ds(e0 + full_ch * _CHUNK, rem)],
                idx_v.at[full_ch, :, pl.ds(0, rem)], ss)

        def idx_drain(i, _):
            pltpu.make_async_copy(
                ei_hbm.at[:, pl.ds(e0 + i * _CHUNK, _CHUNK)],
                idx_v.at[i], ss).wait()
            return 0

        lax.fori_loop(0, full_ch, idx_drain, 0)
        if rem:
            pltpu.make_async_copy(
                ei_hbm.at[:, pl.ds(e0 + full_ch * _CHUNK, rem)],
                idx_v.at[full_ch, :, pl.ds(0, rem)], ss).wait()
        pltpu.make_async_copy(y_hbm.at[pl.ds(yrow0, stage_per_tile)],
                              y_sh.at[pl.ds(yrow0, stage_per_tile)], sg1
                              ).wait()
        for cp in zcopies:
            cp.wait()

        plsc.subcore_barrier()

        def g_start(i, b):
            pltpu.async_copy(y_sh.at[idx_v.at[i, 0]], rows[b], sg[b])

        def g_wait(i, b):
            pltpu.make_async_copy(y_sh.at[idx_v.at[i, 0]], rows[b], sg[b]
                                  ).wait()

        def s_start(i, b):
            pltpu.async_copy(rows[b], acc_sh.at[idx_v.at[i, 1]], sc[b],
                             add=True)

        def s_wait(i, b):
            pltpu.make_async_copy(
                rows[b], acc_sh.at[idx_v.at[i, 1]], sc[b]).wait()

        g_start(0, 0)
        g_start(1, 1)

        def quad_body(g, _):
            for b in range(4):
                i = 4 * g + b
                g_wait(i, b)
                s_start(i, b)
                b2 = (b + 2) % 4
                j = i - 2

                @pl.when(j >= 0)
                def _():
                    s_wait(j, b2)

                @pl.when(i + 2 < nch)
                def _():
                    g_start(i + 2, b2)
            return 0

        lax.fori_loop(0, nch // 4, quad_body, 0)
        s_wait(nch - 2, (nch - 2) % 4)
        s_wait(nch - 1, (nch - 1) % 4)

        plsc.subcore_barrier()

        pltpu.sync_copy(acc_sh.at[pl.ds(row0, rows_per_tile)],
                        acc_out.at[c, pl.ds(row0, rows_per_tile)])

    return pl.kernel(
        body, mesh=mesh, out_type=out_type, scratch_types=scratch,
        compiler_params=pltpu.CompilerParams(use_tc_tiling_on_sc=False))


def _matT(a, w):
    return lax.dot_general(a, w, (((1,), (1,)), ((), ())),
                           preferred_element_type=jnp.float32)


def _pre_body(x_ref, wl_ref, wr_ref, y_ref, r_ref):
    x = x_ref[...]
    yl = _matT(x, wl_ref[...])
    y_ref[...] = jnp.concatenate(
        [yl, jnp.ones((yl.shape[0], _CNTW), jnp.float32)], axis=1)
    r_ref[...] = _matT(x, wr_ref[...])


def _mid_body(acc_ref, r1_ref, b1_ref, wl_ref, wr_ref, emb_ref, y2_ref,
              r2_ref):
    h = r1_ref.shape[1]
    agg = acc_ref[0, :, :h] + acc_ref[1, :, :h]
    cnt = acc_ref[0, :, h:h + 1] + acc_ref[1, :, h:h + 1]
    o = agg / jnp.clip(cnt, 1.0, None) + b1_ref[0] + r1_ref[...]
    nrm = jnp.sqrt(jnp.sum(o * o, axis=1, keepdims=True))
    e = o / jnp.clip(nrm, 1e-12, None)
    emb_ref[...] = e
    hr = jnp.maximum(e, 0.0)
    y2_ref[...] = _matT(hr, wl_ref[...])
    r2_ref[...] = _matT(hr, wr_ref[...])


def _post_body(acc_ref, cnt_ref, r2_ref, b2_ref, out_ref):
    h = cnt_ref.shape[2] - _CNTW
    agg = acc_ref[0] + acc_ref[1]
    cnt = cnt_ref[0, :, h:h + 1] + cnt_ref[1, :, h:h + 1]
    o = agg / jnp.clip(cnt, 1.0, None) + b2_ref[0] + r2_ref[...]
    nrm = jnp.sqrt(jnp.sum(o * o, axis=1, keepdims=True))
    e = o / jnp.clip(nrm, 1e-12, None)
    m = jnp.max(e, axis=1, keepdims=True)
    lse = jnp.log(jnp.sum(jnp.exp(e - m), axis=1, keepdims=True)) + m
    out_ref[...] = e - lse


def kernel(x, edge_index, W1l, b1, W1r, W2l, b2, W2r):
    n, d = x.shape
    e = edge_index.shape[1]
    h = W1l.shape[0]
    o = W2l.shape[0]
    w1 = h + _CNTW

    n_acc = ((n + 1 + _NS * 8 - 1) // (_NS * 8)) * (_NS * 8)
    tile_e = e // _NTILES
    nch = (tile_e + _CHUNK - 1) // _CHUNK
    nch = ((nch + 3) // 4) * 4

    bn = 1000
    y1, r1 = pl.pallas_call(
        _pre_body,
        grid=(n // bn,),
        in_specs=[
            pl.BlockSpec((bn, d), lambda i: (i, 0)),
            pl.BlockSpec((h, d), lambda i: (0, 0)),
            pl.BlockSpec((h, d), lambda i: (0, 0)),
        ],
        out_specs=[
            pl.BlockSpec((bn, w1), lambda i: (i, 0)),
            pl.BlockSpec((bn, h), lambda i: (i, 0)),
        ],
        out_shape=[
            jax.ShapeDtypeStruct((n, w1), jnp.float32),
            jax.ShapeDtypeStruct((n, h), jnp.float32),
        ],
    )(x, W1l, W1r)

    seg1 = _make_sc_segsum(n, n_acc, tile_e, nch, w1)(y1, edge_index)

    emb, y2, r2 = pl.pallas_call(
        _mid_body,
        grid=(n // bn,),
        in_specs=[
            pl.BlockSpec((2, bn, w1), lambda i: (0, i, 0)),
            pl.BlockSpec((bn, h), lambda i: (i, 0)),
            pl.BlockSpec((1, h), lambda i: (0, 0)),
            pl.BlockSpec((o, h), lambda i: (0, 0)),
            pl.BlockSpec((o, h), lambda i: (0, 0)),
        ],
        out_specs=[
            pl.BlockSpec((bn, h), lambda i: (i, 0)),
            pl.BlockSpec((bn, o), lambda i: (i, 0)),
            pl.BlockSpec((bn, o), lambda i: (i, 0)),
        ],
        out_shape=[
            jax.ShapeDtypeStruct((n, h), jnp.float32),
            jax.ShapeDtypeStruct((n, o), jnp.float32),
            jax.ShapeDtypeStruct((n, o), jnp.float32),
        ],
    )(seg1, r1, b1.reshape(1, h), W2l, W2r)

    seg2 = _make_sc_segsum(n, n_acc, tile_e, nch, o)(y2, edge_index)

    out = pl.pallas_call(
        _post_body,
        grid=(n // bn,),
        in_specs=[
            pl.BlockSpec((2, bn, o), lambda i: (0, i, 0)),
            pl.BlockSpec((2, bn, w1), lambda i: (0, i, 0)),
            pl.BlockSpec((bn, o), lambda i: (i, 0)),
            pl.BlockSpec((1, o), lambda i: (0, 0)),
        ],
        out_specs=pl.BlockSpec((bn, o), lambda i: (i, 0)),
        out_shape=jax.ShapeDtypeStruct((n, o), jnp.float32),
    )(seg2, seg1, r2, b2.reshape(1, o))

    return (emb, out)


# speedup vs baseline: 1.0485x; 1.0485x over previous
import functools

import jax
import jax.numpy as jnp
from jax import lax
from jax.experimental import pallas as pl
from jax.experimental.pallas import tpu as pltpu
from jax.experimental.pallas import tpu_sc as plsc

_NC = 2
_NS = 16
_NTILES = _NC * _NS
_CHUNK = 128
_CNTW = 16


def _make_sc_segsum(n_rows, n_acc, tile_e, nch, width):
    rows_per_tile = n_acc // _NS
    stage_per_tile = n_rows // _NS
    full_ch = tile_e // _CHUNK
    rem = tile_e - full_ch * _CHUNK
    mesh = plsc.VectorSubcoreMesh(core_axis_name="c", subcore_axis_name="s")

    out_type = jax.ShapeDtypeStruct((_NC, n_acc, width), jnp.float32)
    scratch = [
        pltpu.VMEM((nch, 2, _CHUNK), jnp.int32),
        pltpu.VMEM((_CHUNK, width), jnp.float32),
        pltpu.VMEM((_CHUNK, width), jnp.float32),
        pltpu.VMEM((_CHUNK, width), jnp.float32),
        pltpu.VMEM((_CHUNK, width), jnp.float32),
        pltpu.VMEM_SHARED((n_rows, width), jnp.float32),
        pltpu.VMEM_SHARED((n_acc, width), jnp.float32),
        pltpu.SemaphoreType.DMA,
        pltpu.SemaphoreType.DMA,
        pltpu.SemaphoreType.DMA,
        pltpu.SemaphoreType.DMA,
        pltpu.SemaphoreType.DMA,
        pltpu.SemaphoreType.DMA,
        pltpu.SemaphoreType.DMA,
        pltpu.SemaphoreType.DMA,
        pltpu.SemaphoreType.DMA,
    ]

    def body(y_hbm, ei_hbm, acc_out,
             idx_v, rows0, rows1, rows2, rows3, y_sh, acc_sh,
             sg0, sg1, sg2, sg3, sc0, sc1, sc2, sc3, ss):
        rows = (rows0, rows1, rows2, rows3)
        sg = (sg0, sg1, sg2, sg3)
        sc = (sc0, sc1, sc2, sc3)
        c = lax.axis_index("c")
        s = lax.axis_index("s")
        tile = c * _NS + s
        e0 = tile * tile_e

        def idx_fire(i, _):
            pltpu.async_copy(
                ei_hbm.at[:, pl.ds(e0 + i * _CHUNK, _CHUNK)], idx_v.at[i], ss)
            return 0

        lax.fori_loop(0, full_ch, idx_fire, 0)

        yrow0 = s * stage_per_tile
        pltpu.async_copy(y_hbm.at[pl.ds(yrow0, stage_per_tile)],
                         y_sh.at[pl.ds(yrow0, stage_per_tile)], sg1)

        z16 = jnp.zeros((16,), jnp.float32)
        wq = width // 16

        def zbody(i, _):
            rows0[i // wq, pl.ds((i % wq) * 16, 16)] = z16
            return 0

        lax.fori_loop(0, _CHUNK * wq, zbody, 0)
        row0 = s * rows_per_tile
        zcopies = []
        off = 0
        k = 0
        while off < rows_per_tile:
            sz = min(_CHUNK, rows_per_tile - off)
            zcopies.append(pltpu.async_copy(
                rows0.at[pl.ds(0, sz)], acc_sh.at[pl.ds(row0 + off, sz)],
                sc[k % 4]))
            off += sz
            k += 1

        dummy = jnp.full((16,), n_rows, jnp.int32)
        zi16 = jnp.zeros((16,), jnp.int32)
        for i in range(full_ch, nch):
            for k in range(_CHUNK // 16):
                idx_v[i, 0, pl.ds(k * 16, 16)] = zi16
                idx_v[i, 1, pl.ds(k * 16, 16)] = dummy
        if rem:
            pltpu.async_copy(
                ei_hbm.at[:, pl.ds(e0 + full_ch * _CHUNK, rem)],
                idx_v.at[full_ch, :, pl.ds(0, rem)], ss)

        def idx_drain(i, _):
            pltpu.make_async_copy(
                ei_hbm.at[:, pl.ds(e0 + i * _CHUNK, _CHUNK)],
                idx_v.at[i], ss).wait()
            return 0

        lax.fori_loop(0, full_ch, idx_drain, 0)
        if rem:
            pltpu.make_async_copy(
                ei_hbm.at[:, pl.ds(e0 + full_ch * _CHUNK, rem)],
                idx_v.at[full_ch, :, pl.ds(0, rem)], ss).wait()
        pltpu.make_async_copy(y_hbm.at[pl.ds(yrow0, stage_per_tile)],
                              y_sh.at[pl.ds(yrow0, stage_per_tile)], sg1
                              ).wait()
        for cp in zcopies:
            cp.wait()

        plsc.subcore_barrier()

        def g_start(i, b):
            pltpu.async_copy(y_sh.at[idx_v.at[i, 0]], rows[b], sg[b])

        def g_wait(i, b):
            pltpu.make_async_copy(y_sh.at[idx_v.at[i, 0]], rows[b], sg[b]
                                  ).wait()

        def s_start(i, b):
            pltpu.async_copy(rows[b], acc_sh.at[idx_v.at[i, 1]], sc[b],
                             add=True)

        def s_wait(i, b):
            pltpu.make_async_copy(
                rows[b], acc_sh.at[idx_v.at[i, 1]], sc[b]).wait()

        g_start(0, 0)
        g_start(1, 1)

        def quad_body(g, _):
            for b in range(4):
                i = 4 * g + b
                g_wait(i, b)
                s_start(i, b)
                b2 = (b + 2) % 4
                j = i - 2

                @pl.when(j >= 0)
                def _():
                    s_wait(j, b2)

                @pl.when(i + 2 < nch)
                def _():
                    g_start(i + 2, b2)
            return 0

        lax.fori_loop(0, nch // 4, quad_body, 0)
        s_wait(nch - 2, (nch - 2) % 4)
        s_wait(nch - 1, (nch - 1) % 4)

        plsc.subcore_barrier()

        pltpu.sync_copy(acc_sh.at[pl.ds(row0, rows_per_tile)],
                        acc_out.at[c, pl.ds(row0, rows_per_tile)])

    return pl.kernel(
        body, mesh=mesh, out_type=out_type, scratch_types=scratch,
        compiler_params=pltpu.CompilerParams(use_tc_tiling_on_sc=False))


def _matT(a, w):
    return lax.dot_general(a, w, (((1,), (1,)), ((), ())),
                           preferred_element_type=jnp.float32)


def _pre_body(x_ref, wl_ref, wr_ref, y_ref, r_ref):
    x = x_ref[...]
    yl = _matT(x, wl_ref[...])
    y_ref[...] = jnp.concatenate(
        [yl, jnp.ones((yl.shape[0], _CNTW), jnp.float32)], axis=1)
    r_ref[...] = _matT(x, wr_ref[...])


def _mid_body(acc_ref, r1_ref, b1_ref, wl_ref, wr_ref, emb_ref, y2_ref,
              r2_ref):
    h = r1_ref.shape[1]
    agg = acc_ref[0, :, :h] + acc_ref[1, :, :h]
    cnt = acc_ref[0, :, h:h + 1] + acc_ref[1, :, h:h + 1]
    o = agg / jnp.clip(cnt, 1.0, None) + b1_ref[0] + r1_ref[...]
    nrm = jnp.sqrt(jnp.sum(o * o, axis=1, keepdims=True))
    e = o / jnp.clip(nrm, 1e-12, None)
    emb_ref[...] = e
    hr = jnp.maximum(e, 0.0)
    y2_ref[...] = _matT(hr, wl_ref[...])
    r2_ref[...] = _matT(hr, wr_ref[...])


def _post_body(acc_ref, cnt_ref, r2_ref, b2_ref, out_ref):
    h = cnt_ref.shape[2] - _CNTW
    agg = acc_ref[0] + acc_ref[1]
    cnt = cnt_ref[0, :, h:h + 1] + cnt_ref[1, :, h:h + 1]
    o = agg / jnp.clip(cnt, 1.0, None) + b2_ref[0] + r2_ref[...]
    nrm = jnp.sqrt(jnp.sum(o * o, axis=1, keepdims=True))
    e = o / jnp.clip(nrm, 1e-12, None)
    m = jnp.max(e, axis=1, keepdims=True)
    lse = jnp.log(jnp.sum(jnp.exp(e - m), axis=1, keepdims=True)) + m
    out_ref[...] = e - lse


def kernel(x, edge_index, W1l, b1, W1r, W2l, b2, W2r):
    n, d = x.shape
    e = edge_index.shape[1]
    h = W1l.shape[0]
    o = W2l.shape[0]
    w1 = h + _CNTW

    n_acc = ((n + 1 + _NS * 8 - 1) // (_NS * 8)) * (_NS * 8)
    tile_e = e // _NTILES
    nch = (tile_e + _CHUNK - 1) // _CHUNK
    nch = ((nch + 3) // 4) * 4

    bn = 2000
    y1, r1 = pl.pallas_call(
        _pre_body,
        grid=(n // bn,),
        in_specs=[
            pl.BlockSpec((bn, d), lambda i: (i, 0)),
            pl.BlockSpec((h, d), lambda i: (0, 0)),
            pl.BlockSpec((h, d), lambda i: (0, 0)),
        ],
        out_specs=[
            pl.BlockSpec((bn, w1), lambda i: (i, 0)),
            pl.BlockSpec((bn, h), lambda i: (i, 0)),
        ],
        out_shape=[
            jax.ShapeDtypeStruct((n, w1), jnp.float32),
            jax.ShapeDtypeStruct((n, h), jnp.float32),
        ],
    )(x, W1l, W1r)

    seg1 = _make_sc_segsum(n, n_acc, tile_e, nch, w1)(y1, edge_index)

    emb, y2, r2 = pl.pallas_call(
        _mid_body,
        grid=(n // bn,),
        in_specs=[
            pl.BlockSpec((2, bn, w1), lambda i: (0, i, 0)),
            pl.BlockSpec((bn, h), lambda i: (i, 0)),
            pl.BlockSpec((1, h), lambda i: (0, 0)),
            pl.BlockSpec((o, h), lambda i: (0, 0)),
            pl.BlockSpec((o, h), lambda i: (0, 0)),
        ],
        out_specs=[
            pl.BlockSpec((bn, h), lambda i: (i, 0)),
            pl.BlockSpec((bn, o), lambda i: (i, 0)),
            pl.BlockSpec((bn, o), lambda i: (i, 0)),
        ],
        out_shape=[
            jax.ShapeDtypeStruct((n, h), jnp.float32),
            jax.ShapeDtypeStruct((n, o), jnp.float32),
            jax.ShapeDtypeStruct((n, o), jnp.float32),
        ],
    )(seg1, r1, b1.reshape(1, h), W2l, W2r)

    seg2 = _make_sc_segsum(n, n_acc, tile_e, nch, o)(y2, edge_index)

    out = pl.pallas_call(
        _post_body,
        grid=(n // bn,),
        in_specs=[
            pl.BlockSpec((2, bn, o), lambda i: (0, i, 0)),
            pl.BlockSpec((2, bn, w1), lambda i: (0, i, 0)),
            pl.BlockSpec((bn, o), lambda i: (i, 0)),
            pl.BlockSpec((1, o), lambda i: (0, 0)),
        ],
        out_specs=pl.BlockSpec((bn, o), lambda i: (i, 0)),
        out_shape=jax.ShapeDtypeStruct((n, o), jnp.float32),
    )(seg2, seg1, r2, b2.reshape(1, o))

    return (emb, out)
